# baseline (speedup 1.0000x reference)
import jax
import jax.numpy as jnp
from jax import lax
from jax.experimental import pallas as pl
from jax.experimental.pallas import tpu as pltpu

B = 4
SEQ = 4096
HID = 2048

CHUNK_OFFS = (0, 1792, 2048)


def _body(pos_ref, hs_ref, w_ref, o_ref, oidx_ref, w_v, tok_v, wsem, gsem):
    for c in range(len(CHUNK_OFFS) - 1):
        lo, hi = CHUNK_OFFS[c], CHUNK_OFFS[c + 1]
        pltpu.make_async_copy(
            w_ref.at[pl.ds(lo, hi - lo)], w_v.at[pl.ds(lo, hi - lo)], wsem.at[c]
        ).start()

    pos = pos_ref[...]
    col = lax.broadcasted_iota(jnp.int32, (B, SEQ), 1)
    key = pos * SEQ + col
    for b in range(B):
        idx = jnp.min(key[b]) & (SEQ - 1)
        oidx_ref[0, b] = idx
        pltpu.make_async_copy(hs_ref.at[b, idx], tok_v.at[b], gsem).start()
    for b in range(B):
        pltpu.make_async_copy(hs_ref.at[b, 0], tok_v.at[b], gsem).wait()

    tok = tok_v[...]
    for c in range(len(CHUNK_OFFS) - 1):
        lo, hi = CHUNK_OFFS[c], CHUNK_OFFS[c + 1]
        pltpu.make_async_copy(
            w_ref.at[pl.ds(lo, hi - lo)], w_v.at[pl.ds(lo, hi - lo)], wsem.at[c]
        ).wait()
        o_ref[:, pl.ds(lo, hi - lo)] = lax.dot_general(
            tok, w_v[pl.ds(lo, hi - lo), :],
            (((1,), (1,)), ((), ())),
            preferred_element_type=jnp.float32,
        )


def _sink_projection(pos, hs, W):
    return pl.pallas_call(
        _body,
        in_specs=[
            pl.BlockSpec((B, SEQ), lambda: (0, 0)),
            pl.BlockSpec(memory_space=pl.ANY),
            pl.BlockSpec(memory_space=pl.ANY),
        ],
        out_specs=[
            pl.BlockSpec((B, HID), lambda: (0, 0)),
            pl.BlockSpec(memory_space=pltpu.SMEM),
        ],
        out_shape=[
            jax.ShapeDtypeStruct((B, HID), jnp.float32),
            jax.ShapeDtypeStruct((1, B), jnp.int32),
        ],
        scratch_shapes=[
            pltpu.VMEM((HID, HID), jnp.float32),
            pltpu.VMEM((B, HID), jnp.float32),
            pltpu.SemaphoreType.DMA((len(CHUNK_OFFS) - 1,)),
            pltpu.SemaphoreType.DMA,
        ],
    )(pos, hs, W)


def kernel(hidden_states, positions, W):
    pos = positions.astype(jnp.int32)
    enhanced, idx = _sink_projection(pos, hidden_states, W)
    return (enhanced, idx[0])


# speedup vs baseline: 1.0570x; 1.0570x over previous
import jax
import jax.numpy as jnp
from jax import lax
from jax.experimental import pallas as pl
from jax.experimental.pallas import tpu as pltpu

B = 4
SEQ = 4096
HID = 2048

CHUNK_OFFS = (0, 1024, 2048)


def _body(pos_ref, hs_ref, w_ref, o_ref, oidx_ref, w_v, tok_v, wsem, gsem):
    for c in range(len(CHUNK_OFFS) - 1):
        lo, hi = CHUNK_OFFS[c], CHUNK_OFFS[c + 1]
        pltpu.make_async_copy(
            w_ref.at[pl.ds(lo, hi - lo)], w_v.at[pl.ds(lo, hi - lo)], wsem.at[c]
        ).start()

    pos = pos_ref[...]
    col = lax.broadcasted_iota(jnp.int32, (B, SEQ), 1)
    key = pos * SEQ + col
    for b in range(B):
        idx = jnp.min(key[b]) & (SEQ - 1)
        oidx_ref[0, b] = idx
        pltpu.make_async_copy(hs_ref.at[b, idx], tok_v.at[b], gsem).start()
    for b in range(B):
        pltpu.make_async_copy(hs_ref.at[b, 0], tok_v.at[b], gsem).wait()

    tok = tok_v[...]
    for c in range(len(CHUNK_OFFS) - 1):
        lo, hi = CHUNK_OFFS[c], CHUNK_OFFS[c + 1]
        pltpu.make_async_copy(
            w_ref.at[pl.ds(lo, hi - lo)], w_v.at[pl.ds(lo, hi - lo)], wsem.at[c]
        ).wait()
        o_ref[:, pl.ds(lo, hi - lo)] = lax.dot_general(
            tok, w_v[pl.ds(lo, hi - lo), :],
            (((1,), (1,)), ((), ())),
            preferred_element_type=jnp.float32,
        )


def _sink_projection(pos, hs, W):
    return pl.pallas_call(
        _body,
        in_specs=[
            pl.BlockSpec((B, SEQ), lambda: (0, 0)),
            pl.BlockSpec(memory_space=pl.ANY),
            pl.BlockSpec(memory_space=pl.ANY),
        ],
        out_specs=[
            pl.BlockSpec((B, HID), lambda: (0, 0)),
            pl.BlockSpec(memory_space=pltpu.SMEM),
        ],
        out_shape=[
            jax.ShapeDtypeStruct((B, HID), jnp.float32),
            jax.ShapeDtypeStruct((1, B), jnp.int32),
        ],
        scratch_shapes=[
            pltpu.VMEM((HID, HID), jnp.float32),
            pltpu.VMEM((B, HID), jnp.float32),
            pltpu.SemaphoreType.DMA((len(CHUNK_OFFS) - 1,)),
            pltpu.SemaphoreType.DMA,
        ],
    )(pos, hs, W)


def kernel(hidden_states, positions, W):
    pos = positions.astype(jnp.int32)
    enhanced, idx = _sink_projection(pos, hidden_states, W)
    return (enhanced, idx[0])
